# baseline (speedup 1.0000x reference)
import jax
import jax.numpy as jnp
from jax import lax
from jax.experimental import pallas as pl
from jax.experimental.pallas import tpu as pltpu
from jax.experimental.pallas import tpu_sc as plsc

NC = 2
NS = 16
NW = NC * NS
NBUF = 2
LANES = 128
C0 = 128
C1 = 72


def kernel(input_ids, weight):
    B, T = input_ids.shape
    V, D = weight.shape
    n = B * T
    assert B % NW == 0 and T == C0 + C1 and D <= LANES
    b_per_w = B // NW
    per_w = b_per_w * T
    flat_ids = input_ids.reshape(n)

    wp = jnp.pad(weight, ((0, 0), (0, LANES - D))) if D < LANES else weight

    mesh = plsc.VectorSubcoreMesh(core_axis_name="c", subcore_axis_name="s")

    scratch = [pltpu.VMEM((per_w,), jnp.int32)]
    scratch += [pltpu.VMEM((C0, LANES), jnp.float32) for _ in range(NBUF)]
    scratch += [pltpu.VMEM((C0, D), jnp.float32) for _ in range(NBUF)]
    scratch += [pltpu.SemaphoreType.DMA for _ in range(2 * NBUF + 1)]

    @pl.kernel(
        out_type=jax.ShapeDtypeStruct((B, T, D), weight.dtype),
        mesh=mesh,
        scratch_types=scratch,
    )
    def gather_kernel(w_hbm, i_hbm, o_hbm, idx_v, *bufs_and_sems):
        rows = bufs_and_sems[:NBUF]
        crows = bufs_and_sems[NBUF:2 * NBUF]
        g_sems = bufs_and_sems[2 * NBUF:3 * NBUF]
        w_sems = bufs_and_sems[3 * NBUF:4 * NBUF]
        i_sem = bufs_and_sems[4 * NBUF]

        wid = lax.axis_index("s") * NC + lax.axis_index("c")
        base_b = wid * b_per_w
        pltpu.async_copy(
            i_hbm.at[pl.ds(wid * per_w, per_w)], idx_v, i_sem).wait()

        cs = (C0, C1)

        def g_copy(lb, b):
            off = lb * T + b * C0
            return pltpu.make_async_copy(
                w_hbm.at[idx_v.at[pl.ds(off, cs[b])]],
                rows[b].at[pl.ds(0, cs[b])], g_sems[b])

        def compact(b):
            @pl.loop(0, cs[b])
            def _(j):
                for g in range(D // 16):
                    slc = (j, pl.ds(g * 16, 16))
                    crows[b].at[slc][...] = rows[b].at[slc][...]

        def w_copy(lb, b):
            return pltpu.make_async_copy(
                crows[b].at[pl.ds(0, cs[b])],
                o_hbm.at[base_b + lb, pl.ds(b * C0, cs[b]), :], w_sems[b])

        for b in range(NBUF):
            g_copy(0, b).start()

        @pl.loop(0, b_per_w)
        def _(lb):
            for b in range(NBUF):
                g_copy(lb, b).wait()
                compact(b)
                w_copy(lb, b).start()
                w_copy(lb, b).wait()

                @pl.when(lb + 1 < b_per_w)
                def _():
                    g_copy(lb + 1, b).start()

    return gather_kernel(wp, flat_ids)


# speedup vs baseline: 1.1771x; 1.1771x over previous
import jax
import jax.numpy as jnp
from jax import lax
from jax.experimental import pallas as pl
from jax.experimental.pallas import tpu as pltpu
from jax.experimental.pallas import tpu_sc as plsc

NC = 2
NS = 16
NW = NC * NS
CHUNK = 128
NBUF = 5
LANES = 128


def kernel(input_ids, weight):
    B, T = input_ids.shape
    V, D = weight.shape
    n = B * T
    assert n % (NW * CHUNK) == 0 and D <= LANES
    per_w = n // NW
    steps = per_w // CHUNK
    flat_ids = input_ids.reshape(n)

    wp = jnp.pad(weight, ((0, 0), (0, LANES - D))) if D < LANES else weight

    mesh = plsc.VectorSubcoreMesh(core_axis_name="c", subcore_axis_name="s")

    scratch = [pltpu.VMEM((per_w,), jnp.int32)]
    scratch += [pltpu.VMEM((CHUNK, LANES), jnp.float32) for _ in range(NBUF)]
    scratch += [pltpu.SemaphoreType.DMA for _ in range(2 * NBUF + 1)]

    @pl.kernel(
        out_type=jax.ShapeDtypeStruct((n, LANES), weight.dtype),
        mesh=mesh,
        scratch_types=scratch,
    )
    def gather_kernel(w_hbm, i_hbm, o_hbm, idx_v, *bufs_and_sems):
        rows = bufs_and_sems[:NBUF]
        g_sems = bufs_and_sems[NBUF:2 * NBUF]
        w_sems = bufs_and_sems[2 * NBUF:3 * NBUF]
        i_sem = bufs_and_sems[3 * NBUF]

        wid = lax.axis_index("s") * NC + lax.axis_index("c")
        base = wid * per_w
        pltpu.async_copy(i_hbm.at[pl.ds(base, per_w)], idx_v, i_sem).wait()

        def g_copy(c, b):
            return pltpu.make_async_copy(
                w_hbm.at[idx_v.at[pl.ds(c * CHUNK, CHUNK)]], rows[b], g_sems[b])

        def w_copy(c, b):
            return pltpu.make_async_copy(
                rows[b], o_hbm.at[pl.ds(base + c * CHUNK, CHUNK)], w_sems[b])

        for b in range(NBUF):
            g_copy(b, b).start()

        @pl.loop(0, steps // NBUF)
        def _(o):
            for b in range(NBUF):
                c = o * NBUF + b
                g_copy(c, b).wait()
                w_copy(c, b).start()
                w_copy(c, b).wait()

                @pl.when(c + NBUF < steps)
                def _():
                    g_copy(c + NBUF, b).start()

    out = gather_kernel(wp, flat_ids)
    return out.reshape(B, T, LANES)[:, :, :D]
